# Initial kernel scaffold; baseline (speedup 1.0000x reference)
import jax
import jax.numpy as jnp
from jax.experimental import pallas as pl


def kernel(features, edge_index, W):
    raise NotImplementedError("write your pallas kernel here")


# speedup vs baseline: 191.2002x; 191.2002x over previous
import functools

import jax
import jax.numpy as jnp
from jax import lax
from jax.experimental import pallas as pl
from jax.experimental.pallas import tpu as pltpu
from jax.experimental.pallas import tpu_sc as plsc

ROW = 32
NC, NS = 2, 16
NW = NC * NS
CH = 400


def _sc_segment_sum(featpad, src, dst, zeros, n, e):
    edges_per_tile = e // NW
    nchunk = edges_per_tile // CH
    npad = -(-n // (NS * 8)) * (NS * 8)
    rows_per_sub = npad // NS

    mesh = plsc.VectorSubcoreMesh(core_axis_name="c", subcore_axis_name="s")

    @functools.partial(
        pl.kernel,
        out_type=jax.ShapeDtypeStruct((NC, npad, ROW), jnp.float32),
        mesh=mesh,
        scratch_types=[
            pltpu.VMEM_SHARED((npad, ROW), jnp.float32),
            pltpu.VMEM((CH,), jnp.int32),
            pltpu.VMEM((CH,), jnp.int32),
            pltpu.VMEM((CH, ROW), jnp.float32),
            pltpu.SemaphoreType.DMA,
        ],
        compiler_params=pltpu.CompilerParams(use_tc_tiling_on_sc=False),
    )
    def sc_kernel(featpad_hbm, src_hbm, dst_hbm, zeros_hbm, out_hbm,
                  acc_sh, sidx, didx, rows, sem):
        cid = lax.axis_index("c")
        sid = lax.axis_index("s")
        wid = sid * NC + cid

        pltpu.sync_copy(zeros_hbm,
                        acc_sh.at[pl.ds(sid * rows_per_sub, rows_per_sub)])
        plsc.subcore_barrier()

        base_edge = wid * edges_per_tile

        def chunk(i, carry):
            off = base_edge + i * CH
            pltpu.sync_copy(src_hbm.at[pl.ds(off, CH)], sidx)
            pltpu.sync_copy(dst_hbm.at[pl.ds(off, CH)], didx)
            pltpu.async_copy(featpad_hbm.at[sidx], rows, sem).wait()
            pltpu.sync_copy(rows, acc_sh.at[didx], add=True)
            return carry

        lax.fori_loop(0, nchunk, chunk, 0)

        plsc.subcore_barrier()
        sl = pl.ds(sid * rows_per_sub, rows_per_sub)
        pltpu.sync_copy(acc_sh.at[sl], out_hbm.at[cid, sl])

    return sc_kernel(featpad, src, dst, zeros)


def _tc_combine(partials, featpad, m, n):
    nb = 2000
    grid = n // nb

    def body(part_ref, feat_ref, m_ref, out_ref):
        p = part_ref[0] + part_ref[1]
        cnt = jnp.maximum(p[:, 24:25], 1.0)
        mean = p[:, :24] / cnt
        x = jnp.concatenate([feat_ref[:, :24], mean], axis=1)
        out_ref[...] = jnp.dot(x, m_ref[...],
                               preferred_element_type=jnp.float32)

    return pl.pallas_call(
        body,
        grid=(grid,),
        in_specs=[
            pl.BlockSpec((2, nb, ROW), lambda i: (0, i, 0)),
            pl.BlockSpec((nb, ROW), lambda i: (i, 0)),
            pl.BlockSpec((48, 480), lambda i: (0, 0)),
        ],
        out_specs=pl.BlockSpec((nb, 480), lambda i: (i, 0)),
        out_shape=jax.ShapeDtypeStruct((n, 480), jnp.float32),
        compiler_params=pltpu.CompilerParams(
            dimension_semantics=("parallel",)),
    )(partials, featpad, m)


def kernel(features, edge_index, W):
    n, b, t, in_feat = features.shape
    e = edge_index.shape[1]
    bt = b * t

    feat24 = features.reshape(n, bt * in_feat)
    featpad = jnp.concatenate(
        [feat24,
         jnp.ones((n, 1), jnp.float32),
         jnp.zeros((n, ROW - bt * in_feat - 1), jnp.float32)], axis=1)

    npad = -(-n // (NS * 8)) * (NS * 8)
    zeros = jnp.zeros((npad // NS, ROW), jnp.float32)

    partials = _sc_segment_sum(featpad, edge_index[1], edge_index[0],
                               zeros, n, e)

    w = W.reshape(-1)
    out_feat = w.shape[0]
    eye = jnp.eye(bt, dtype=jnp.float32)
    wtop = jnp.concatenate([w, jnp.zeros((out_feat,), jnp.float32)])
    wbot = jnp.concatenate([jnp.zeros((out_feat,), jnp.float32), w])
    m = jnp.concatenate(
        [jnp.kron(eye, wtop.reshape(1, 2 * out_feat)),
         jnp.kron(eye, wbot.reshape(1, 2 * out_feat))], axis=0)

    out = _tc_combine(partials, featpad, m, n)
    return out.reshape(n, b, t, 2 * out_feat)


# speedup vs baseline: 218.4306x; 1.1424x over previous
import functools

import jax
import jax.numpy as jnp
from jax import lax
from jax.experimental import pallas as pl
from jax.experimental.pallas import tpu as pltpu
from jax.experimental.pallas import tpu_sc as plsc

ROW = 32
NC, NS = 2, 16
NW = NC * NS
CH = 200


def _sc_segment_sum(featpad, src, dst, zeros, n, e):
    edges_per_tile = e // NW
    nchunk = edges_per_tile // CH
    npad = -(-n // (NS * 8)) * (NS * 8)
    rows_per_sub = npad // NS

    mesh = plsc.VectorSubcoreMesh(core_axis_name="c", subcore_axis_name="s")

    @functools.partial(
        pl.kernel,
        out_type=[jax.ShapeDtypeStruct((npad, ROW), jnp.float32),
                  jax.ShapeDtypeStruct((npad, ROW), jnp.float32)],
        mesh=mesh,
        scratch_types=[
            pltpu.VMEM_SHARED((npad, ROW), jnp.float32),
            pltpu.VMEM((CH,), jnp.int32),
            pltpu.VMEM((CH,), jnp.int32),
            pltpu.VMEM((CH, ROW), jnp.float32),
            pltpu.VMEM((CH,), jnp.int32),
            pltpu.VMEM((CH,), jnp.int32),
            pltpu.VMEM((CH, ROW), jnp.float32),
            pltpu.SemaphoreType.DMA,
            pltpu.SemaphoreType.DMA,
            pltpu.SemaphoreType.DMA,
            pltpu.SemaphoreType.DMA,
        ],
        compiler_params=pltpu.CompilerParams(use_tc_tiling_on_sc=False),
    )
    def sc_kernel(featpad_hbm, src_hbm, dst_hbm, zeros_hbm, out0_hbm, out1_hbm,
                  acc_sh, sidx0, didx0, rows0, sidx1, didx1, rows1,
                  sg0, sg1, si0, si1):
        cid = lax.axis_index("c")
        sid = lax.axis_index("s")
        wid = sid * NC + cid

        sidx = (sidx0, sidx1)
        didx = (didx0, didx1)
        rows = (rows0, rows1)
        sg = (sg0, sg1)
        si = (si0, si1)

        pltpu.sync_copy(zeros_hbm,
                        acc_sh.at[pl.ds(sid * rows_per_sub, rows_per_sub)])
        plsc.subcore_barrier()

        base_edge = wid * edges_per_tile
        max_off = e - CH

        def idx_start(j, b):
            off = jnp.minimum(base_edge + j * CH, max_off)
            pltpu.async_copy(src_hbm.at[pl.ds(off, CH)], sidx[b], si[b])
            pltpu.async_copy(dst_hbm.at[pl.ds(off, CH)], didx[b], si[b])

        def idx_wait(b):
            pltpu.make_async_copy(src_hbm.at[pl.ds(0, CH)], sidx[b],
                                  si[b]).wait()
            pltpu.make_async_copy(dst_hbm.at[pl.ds(0, CH)], didx[b],
                                  si[b]).wait()

        def gather_start(b):
            pltpu.async_copy(featpad_hbm.at[sidx[b]], rows[b], sg[b])

        def gather_wait(b):
            pltpu.make_async_copy(featpad_hbm.at[sidx[b]], rows[b],
                                  sg[b]).wait()

        idx_start(0, 0)
        idx_start(1, 1)

        def pair(i, carry):
            j = 2 * i
            idx_wait(0)
            gather_start(0)
            idx_wait(1)
            gather_start(1)
            gather_wait(0)
            pltpu.sync_copy(rows[0], acc_sh.at[didx[0]], add=True)
            idx_start(j + 2, 0)
            gather_wait(1)
            pltpu.sync_copy(rows[1], acc_sh.at[didx[1]], add=True)
            idx_start(j + 3, 1)
            return carry

        lax.fori_loop(0, nchunk // 2, pair, 0)
        idx_wait(0)
        idx_wait(1)

        plsc.subcore_barrier()
        sl = pl.ds(sid * rows_per_sub, rows_per_sub)

        @pl.when(cid == 0)
        def _():
            pltpu.sync_copy(acc_sh.at[sl], out0_hbm.at[sl])

        @pl.when(cid == 1)
        def _():
            pltpu.sync_copy(acc_sh.at[sl], out1_hbm.at[sl])

    return sc_kernel(featpad, src, dst, zeros)


def _tc_combine(p0, p1, featpad, m, n):
    nb = 2000
    grid = n // nb

    def body(p0_ref, p1_ref, feat_ref, m_ref, out_ref):
        p = p0_ref[...] + p1_ref[...]
        cnt = jnp.maximum(p[:, 24:25], 1.0)
        mean = p[:, :24] / cnt
        x = jnp.concatenate([feat_ref[:, :24], mean], axis=1)
        out_ref[...] = jnp.dot(x, m_ref[...],
                               preferred_element_type=jnp.float32)

    return pl.pallas_call(
        body,
        grid=(grid,),
        in_specs=[
            pl.BlockSpec((nb, ROW), lambda i: (i, 0)),
            pl.BlockSpec((nb, ROW), lambda i: (i, 0)),
            pl.BlockSpec((nb, ROW), lambda i: (i, 0)),
            pl.BlockSpec((48, 480), lambda i: (0, 0)),
        ],
        out_specs=pl.BlockSpec((nb, 480), lambda i: (i, 0)),
        out_shape=jax.ShapeDtypeStruct((n, 480), jnp.float32),
        compiler_params=pltpu.CompilerParams(
            dimension_semantics=("parallel",)),
    )(p0, p1, featpad, m)


def kernel(features, edge_index, W):
    n, b, t, in_feat = features.shape
    e = edge_index.shape[1]
    bt = b * t

    feat24 = features.reshape(n, bt * in_feat)
    featpad = jnp.concatenate(
        [feat24,
         jnp.ones((n, 1), jnp.float32),
         jnp.zeros((n, ROW - bt * in_feat - 1), jnp.float32)], axis=1)

    npad = -(-n // (NS * 8)) * (NS * 8)
    zeros = jnp.zeros((npad // NS, ROW), jnp.float32)

    p0, p1 = _sc_segment_sum(featpad, edge_index[1], edge_index[0],
                             zeros, n, e)

    w = W.reshape(-1)
    out_feat = w.shape[0]
    eye = jnp.eye(bt, dtype=jnp.float32)
    wtop = jnp.concatenate([w, jnp.zeros((out_feat,), jnp.float32)])
    wbot = jnp.concatenate([jnp.zeros((out_feat,), jnp.float32), w])
    m = jnp.concatenate(
        [jnp.kron(eye, wtop.reshape(1, 2 * out_feat)),
         jnp.kron(eye, wbot.reshape(1, 2 * out_feat))], axis=0)

    out = _tc_combine(p0, p1, featpad, m, n)
    return out.reshape(n, b, t, 2 * out_feat)


# speedup vs baseline: 353.1352x; 1.6167x over previous
import functools

import jax
import jax.numpy as jnp
from jax import lax
from jax.experimental import pallas as pl
from jax.experimental.pallas import tpu as pltpu
from jax.experimental.pallas import tpu_sc as plsc

ROW = 32
NC, NS = 2, 16
NW = NC * NS
CH = 200
NBLK = 2048


def _tc_build_table(featc, n, bt):
    def body(fc_ref, tab_ref):
        ft = fc_ref[...]
        t32 = jnp.concatenate(
            [ft, jnp.ones((1, NBLK), jnp.float32),
             jnp.zeros((ROW - bt - 1, NBLK), jnp.float32)], axis=0)
        tab_ref[...] = jnp.transpose(t32)

    return pl.pallas_call(
        body,
        grid=(-(-n // NBLK),),
        in_specs=[pl.BlockSpec((bt, NBLK), lambda i: (0, i))],
        out_specs=pl.BlockSpec((NBLK, ROW), lambda i: (i, 0)),
        out_shape=jax.ShapeDtypeStruct((n, ROW), jnp.float32),
        compiler_params=pltpu.CompilerParams(
            dimension_semantics=("parallel",)),
    )(featc)


def _sc_segment_sum(table, src, dst, zeros, n, e):
    edges_per_tile = e // NW
    nchunk = edges_per_tile // CH
    npad = -(-n // (NS * 8)) * (NS * 8)
    rows_per_sub = npad // NS

    mesh = plsc.VectorSubcoreMesh(core_axis_name="c", subcore_axis_name="s")

    @functools.partial(
        pl.kernel,
        out_type=[jax.ShapeDtypeStruct((npad, ROW), jnp.float32),
                  jax.ShapeDtypeStruct((npad, ROW), jnp.float32)],
        mesh=mesh,
        scratch_types=[
            pltpu.VMEM_SHARED((npad, ROW), jnp.float32),
            pltpu.VMEM((CH,), jnp.int32),
            pltpu.VMEM((CH,), jnp.int32),
            pltpu.VMEM((CH, ROW), jnp.float32),
            pltpu.VMEM((CH,), jnp.int32),
            pltpu.VMEM((CH,), jnp.int32),
            pltpu.VMEM((CH, ROW), jnp.float32),
            pltpu.SemaphoreType.DMA,
            pltpu.SemaphoreType.DMA,
            pltpu.SemaphoreType.DMA,
            pltpu.SemaphoreType.DMA,
        ],
        compiler_params=pltpu.CompilerParams(use_tc_tiling_on_sc=False),
    )
    def sc_kernel(table_hbm, src_hbm, dst_hbm, zeros_hbm, out0_hbm, out1_hbm,
                  acc_sh, sidx0, didx0, rows0, sidx1, didx1, rows1,
                  sg0, sg1, si0, si1):
        cid = lax.axis_index("c")
        sid = lax.axis_index("s")
        wid = sid * NC + cid

        sidx = (sidx0, sidx1)
        didx = (didx0, didx1)
        rows = (rows0, rows1)
        sg = (sg0, sg1)
        si = (si0, si1)

        pltpu.sync_copy(zeros_hbm,
                        acc_sh.at[pl.ds(sid * rows_per_sub, rows_per_sub)])
        plsc.subcore_barrier()

        base_edge = wid * edges_per_tile
        max_off = e - CH

        def idx_start(j, b):
            off = jnp.minimum(base_edge + j * CH, max_off)
            pltpu.async_copy(src_hbm.at[pl.ds(off, CH)], sidx[b], si[b])
            pltpu.async_copy(dst_hbm.at[pl.ds(off, CH)], didx[b], si[b])

        def idx_wait(b):
            pltpu.make_async_copy(src_hbm.at[pl.ds(0, CH)], sidx[b],
                                  si[b]).wait()
            pltpu.make_async_copy(dst_hbm.at[pl.ds(0, CH)], didx[b],
                                  si[b]).wait()

        def gather_start(b):
            pltpu.async_copy(table_hbm.at[sidx[b]], rows[b], sg[b])

        def gather_wait(b):
            pltpu.make_async_copy(table_hbm.at[sidx[b]], rows[b],
                                  sg[b]).wait()

        idx_start(0, 0)
        idx_start(1, 1)

        def pair(i, carry):
            j = 2 * i
            idx_wait(0)
            gather_start(0)
            idx_wait(1)
            gather_start(1)
            gather_wait(0)
            pltpu.sync_copy(rows[0], acc_sh.at[didx[0]], add=True)
            idx_start(j + 2, 0)
            gather_wait(1)
            pltpu.sync_copy(rows[1], acc_sh.at[didx[1]], add=True)
            idx_start(j + 3, 1)
            return carry

        lax.fori_loop(0, nchunk // 2, pair, 0)
        idx_wait(0)
        idx_wait(1)

        plsc.subcore_barrier()
        sl = pl.ds(sid * rows_per_sub, rows_per_sub)

        @pl.when(cid == 0)
        def _():
            pltpu.sync_copy(acc_sh.at[sl], out0_hbm.at[sl])

        @pl.when(cid == 1)
        def _():
            pltpu.sync_copy(acc_sh.at[sl], out1_hbm.at[sl])

    return sc_kernel(table, src, dst, zeros)


def _tc_combine(p0, p1, featc, pmat, n, b, t, of2):
    bt = b * t

    def body(p0_ref, p1_ref, fc_ref, pm_ref, out_ref):
        p = p0_ref[...] + p1_ref[...]
        pt = jnp.transpose(p)
        cnt = jnp.maximum(pt[bt:bt + 1, :], 1.0)
        mean = pt[:bt, :] / cnt
        x = jnp.concatenate([fc_ref[...], mean], axis=0)
        y = jnp.dot(pm_ref[...], x,
                    preferred_element_type=jnp.float32)
        out_ref[...] = y.reshape(t, of2, b, NBLK)

    return pl.pallas_call(
        body,
        grid=(-(-n // NBLK),),
        in_specs=[
            pl.BlockSpec((NBLK, ROW), lambda i: (i, 0)),
            pl.BlockSpec((NBLK, ROW), lambda i: (i, 0)),
            pl.BlockSpec((bt, NBLK), lambda i: (0, i)),
            pl.BlockSpec((t * of2 * b, 2 * bt), lambda i: (0, 0)),
        ],
        out_specs=pl.BlockSpec((t, of2, b, NBLK), lambda i: (0, 0, 0, i)),
        out_shape=jax.ShapeDtypeStruct((t, of2, b, n), jnp.float32),
        compiler_params=pltpu.CompilerParams(
            dimension_semantics=("parallel",)),
    )(p0, p1, featc, pmat)


def kernel(features, edge_index, W):
    n, b, t, in_feat = features.shape
    e = edge_index.shape[1]
    bt = b * t

    featc = jnp.transpose(features, (1, 2, 3, 0)).reshape(bt, n)

    table = _tc_build_table(featc, n, bt)

    npad = -(-n // (NS * 8)) * (NS * 8)
    zeros = jnp.zeros((npad // NS, ROW), jnp.float32)

    p0, p1 = _sc_segment_sum(table, edge_index[1], edge_index[0],
                             zeros, n, e)

    w = W.reshape(-1)
    of = w.shape[0]
    of2 = 2 * of
    kk = jnp.arange(bt)
    tt = kk % t
    bb = kk // t
    r_feat = tt * (of2 * b) + jnp.arange(of)[:, None] * b + bb[None, :]
    r_mean = r_feat + of * b
    pmat = jnp.zeros((t * of2 * b, 2 * bt), jnp.float32)
    pmat = pmat.at[r_feat.reshape(-1),
                   jnp.tile(kk, (of,))].set(jnp.repeat(w, bt))
    pmat = pmat.at[r_mean.reshape(-1),
                   jnp.tile(kk + bt, (of,))].set(jnp.repeat(w, bt))

    out_p = _tc_combine(p0, p1, featc, pmat, n, b, t, of2)
    return jnp.transpose(out_p, (3, 2, 0, 1))


# speedup vs baseline: 373.3349x; 1.0572x over previous
import functools

import jax
import jax.numpy as jnp
from jax import lax
from jax.experimental import pallas as pl
from jax.experimental.pallas import tpu as pltpu
from jax.experimental.pallas import tpu_sc as plsc

ROW = 32
NC, NS = 2, 16
NW = NC * NS
CH = 200
NBLK = 2048


def _tc_build_table(featc, n, bt):
    def body(fc_ref, tab_ref):
        ft = fc_ref[...]
        t32 = jnp.concatenate(
            [ft, jnp.ones((1, NBLK), jnp.float32),
             jnp.zeros((ROW - bt - 1, NBLK), jnp.float32)], axis=0)
        tab_ref[...] = jnp.transpose(t32)

    return pl.pallas_call(
        body,
        grid=(-(-n // NBLK),),
        in_specs=[pl.BlockSpec((bt, NBLK), lambda i: (0, i))],
        out_specs=pl.BlockSpec((NBLK, ROW), lambda i: (i, 0)),
        out_shape=jax.ShapeDtypeStruct((n, ROW), jnp.float32),
        compiler_params=pltpu.CompilerParams(
            dimension_semantics=("parallel",)),
    )(featc)


def _sc_segment_sum(table, src, dst, zeros, n, e):
    edges_per_tile = e // NW
    nchunk = edges_per_tile // CH
    npad = -(-n // (NS * 8)) * (NS * 8)
    rows_per_sub = npad // NS

    mesh = plsc.VectorSubcoreMesh(core_axis_name="c", subcore_axis_name="s")

    nd = 4

    @functools.partial(
        pl.kernel,
        out_type=[jax.ShapeDtypeStruct((npad, ROW), jnp.float32),
                  jax.ShapeDtypeStruct((npad, ROW), jnp.float32)],
        mesh=mesh,
        scratch_types=(
            [pltpu.VMEM_SHARED((npad, ROW), jnp.float32)]
            + [pltpu.VMEM((CH,), jnp.int32) for _ in range(2 * nd)]
            + [pltpu.VMEM((CH, ROW), jnp.float32) for _ in range(nd)]
            + [pltpu.SemaphoreType.DMA for _ in range(3 * nd)]
        ),
        compiler_params=pltpu.CompilerParams(use_tc_tiling_on_sc=False),
    )
    def sc_kernel(table_hbm, src_hbm, dst_hbm, zeros_hbm, out0_hbm, out1_hbm,
                  acc_sh, *bufs):
        sidx = bufs[0:nd]
        didx = bufs[nd:2 * nd]
        rows = bufs[2 * nd:3 * nd]
        sg = bufs[3 * nd:4 * nd]
        si = bufs[4 * nd:5 * nd]
        ss = bufs[5 * nd:6 * nd]

        cid = lax.axis_index("c")
        sid = lax.axis_index("s")
        wid = sid * NC + cid

        pltpu.sync_copy(zeros_hbm,
                        acc_sh.at[pl.ds(sid * rows_per_sub, rows_per_sub)])
        plsc.subcore_barrier()

        base_edge = wid * edges_per_tile
        max_off = e - CH

        def idx_start(j, b):
            off = jnp.minimum(base_edge + j * CH, max_off)
            pltpu.async_copy(src_hbm.at[pl.ds(off, CH)], sidx[b], si[b])
            pltpu.async_copy(dst_hbm.at[pl.ds(off, CH)], didx[b], si[b])

        def idx_wait(b):
            pltpu.make_async_copy(src_hbm.at[pl.ds(0, CH)], sidx[b],
                                  si[b]).wait()
            pltpu.make_async_copy(dst_hbm.at[pl.ds(0, CH)], didx[b],
                                  si[b]).wait()

        def gather_start(b):
            pltpu.async_copy(table_hbm.at[sidx[b]], rows[b], sg[b])

        def gather_wait(b):
            pltpu.make_async_copy(table_hbm.at[sidx[b]], rows[b],
                                  sg[b]).wait()

        def scatter_start(b):
            pltpu.async_copy(rows[b], acc_sh.at[didx[b]], ss[b], add=True)

        def scatter_wait(b):
            pltpu.make_async_copy(rows[b], acc_sh.at[didx[b]], ss[b]).wait()

        for b in (2, 3):
            pltpu.sync_copy(zeros_hbm.at[pl.ds(0, CH)], rows[b])
            pltpu.sync_copy(dst_hbm.at[pl.ds(0, CH)], didx[b])
            scatter_start(b)
        idx_start(0, 0)
        idx_start(1, 1)
        idx_wait(0)
        gather_start(0)

        def quad(i, carry):
            for k in range(nd):
                c = nd * i + k
                b, b1, b2 = k, (k + 1) % nd, (k + 2) % nd
                idx_wait(b1)
                gather_start(b1)
                gather_wait(b)
                scatter_start(b)
                scatter_wait(b2)
                idx_start(c + 2, b2)
            return carry

        nloop = (nchunk - 2) // nd
        lax.fori_loop(0, nloop, quad, 0)

        idx_wait(1)
        gather_start(1)
        gather_wait(0)
        scatter_start(0)
        gather_wait(1)
        scatter_start(1)
        for b in range(nd):
            scatter_wait(b)

        plsc.subcore_barrier()
        sl = pl.ds(sid * rows_per_sub, rows_per_sub)

        @pl.when(cid == 0)
        def _():
            pltpu.sync_copy(acc_sh.at[sl], out0_hbm.at[sl])

        @pl.when(cid == 1)
        def _():
            pltpu.sync_copy(acc_sh.at[sl], out1_hbm.at[sl])

    return sc_kernel(table, src, dst, zeros)


def _tc_combine(p0, p1, featc, pmat, n, b, t, of2):
    bt = b * t

    def body(p0_ref, p1_ref, fc_ref, pm_ref, out_ref):
        p = p0_ref[...] + p1_ref[...]
        pt = jnp.transpose(p)
        cnt = jnp.maximum(pt[bt:bt + 1, :], 1.0)
        mean = pt[:bt, :] / cnt
        x = jnp.concatenate([fc_ref[...], mean], axis=0)
        y = jnp.dot(pm_ref[...], x,
                    preferred_element_type=jnp.float32)
        out_ref[...] = y.reshape(t, of2, b, NBLK)

    return pl.pallas_call(
        body,
        grid=(-(-n // NBLK),),
        in_specs=[
            pl.BlockSpec((NBLK, ROW), lambda i: (i, 0)),
            pl.BlockSpec((NBLK, ROW), lambda i: (i, 0)),
            pl.BlockSpec((bt, NBLK), lambda i: (0, i)),
            pl.BlockSpec((t * of2 * b, 2 * bt), lambda i: (0, 0)),
        ],
        out_specs=pl.BlockSpec((t, of2, b, NBLK), lambda i: (0, 0, 0, i)),
        out_shape=jax.ShapeDtypeStruct((t, of2, b, n), jnp.float32),
        compiler_params=pltpu.CompilerParams(
            dimension_semantics=("parallel",)),
    )(p0, p1, featc, pmat)


def kernel(features, edge_index, W):
    n, b, t, in_feat = features.shape
    e = edge_index.shape[1]
    bt = b * t

    featc = jnp.transpose(features, (1, 2, 3, 0)).reshape(bt, n)

    table = _tc_build_table(featc, n, bt)

    npad = -(-n // (NS * 8)) * (NS * 8)
    zeros = jnp.zeros((npad // NS, ROW), jnp.float32)

    p0, p1 = _sc_segment_sum(table, edge_index[1], edge_index[0],
                             zeros, n, e)

    w = W.reshape(-1)
    of = w.shape[0]
    of2 = 2 * of
    kk = jnp.arange(bt)
    tt = kk % t
    bb = kk // t
    r_feat = tt * (of2 * b) + jnp.arange(of)[:, None] * b + bb[None, :]
    r_mean = r_feat + of * b
    pmat = jnp.zeros((t * of2 * b, 2 * bt), jnp.float32)
    pmat = pmat.at[r_feat.reshape(-1),
                   jnp.tile(kk, (of,))].set(jnp.repeat(w, bt))
    pmat = pmat.at[r_mean.reshape(-1),
                   jnp.tile(kk + bt, (of,))].set(jnp.repeat(w, bt))

    out_p = _tc_combine(p0, p1, featc, pmat, n, b, t, of2)
    return jnp.transpose(out_p, (3, 2, 0, 1))


# speedup vs baseline: 490.4749x; 1.3138x over previous
import functools

import jax
import jax.numpy as jnp
from jax import lax
from jax.experimental import pallas as pl
from jax.experimental.pallas import tpu as pltpu
from jax.experimental.pallas import tpu_sc as plsc

ROW = 32
NC, NS = 2, 16
NW = NC * NS
CH = 200
NBLK = 2048


def _tc_prep(feat4, edges3, n, b, t):
    bt = b * t
    grid = -(-n // NBLK)
    erows = edges3.shape[0]
    eblk = -(-erows // (grid * 8)) * 8

    def body(f_ref, e_ref, tab_ref, fc_ref, d_ref, s_ref):
        ft = f_ref[...].reshape(bt, NBLK)
        fc_ref[...] = ft
        t32 = jnp.concatenate(
            [ft, jnp.ones((1, NBLK), jnp.float32),
             jnp.zeros((ROW - bt - 1, NBLK), jnp.float32)], axis=0)
        tab_ref[...] = jnp.transpose(t32)
        e3 = e_ref[...]
        d_ref[...] = e3[:, 0, :]
        s_ref[...] = e3[:, 1, :]

    return pl.pallas_call(
        body,
        grid=(grid,),
        in_specs=[
            pl.BlockSpec((b, t, 1, NBLK), lambda i: (0, 0, 0, i)),
            pl.BlockSpec((eblk, 2, 128), lambda i: (i, 0, 0)),
        ],
        out_specs=[
            pl.BlockSpec((NBLK, ROW), lambda i: (i, 0)),
            pl.BlockSpec((bt, NBLK), lambda i: (0, i)),
            pl.BlockSpec((eblk, 128), lambda i: (i, 0)),
            pl.BlockSpec((eblk, 128), lambda i: (i, 0)),
        ],
        out_shape=[
            jax.ShapeDtypeStruct((n, ROW), jnp.float32),
            jax.ShapeDtypeStruct((bt, n), jnp.float32),
            jax.ShapeDtypeStruct((erows, 128), jnp.int32),
            jax.ShapeDtypeStruct((erows, 128), jnp.int32),
        ],
        compiler_params=pltpu.CompilerParams(
            dimension_semantics=("arbitrary",)),
    )(feat4, edges3)


def _sc_segment_sum(table, src, dst, zeros, n, e):
    edges_per_tile = e // NW
    nchunk = edges_per_tile // CH
    npad = -(-n // (NS * 8)) * (NS * 8)
    rows_per_sub = npad // NS

    mesh = plsc.VectorSubcoreMesh(core_axis_name="c", subcore_axis_name="s")

    nd = 4

    @functools.partial(
        pl.kernel,
        out_type=[jax.ShapeDtypeStruct((npad, ROW), jnp.float32),
                  jax.ShapeDtypeStruct((npad, ROW), jnp.float32)],
        mesh=mesh,
        scratch_types=(
            [pltpu.VMEM_SHARED((npad, ROW), jnp.float32)]
            + [pltpu.VMEM((CH,), jnp.int32) for _ in range(2 * nd)]
            + [pltpu.VMEM((CH, ROW), jnp.float32) for _ in range(nd)]
            + [pltpu.SemaphoreType.DMA for _ in range(3 * nd)]
        ),
        compiler_params=pltpu.CompilerParams(use_tc_tiling_on_sc=False),
    )
    def sc_kernel(table_hbm, src_hbm, dst_hbm, zeros_hbm, out0_hbm, out1_hbm,
                  acc_sh, *bufs):
        sidx = bufs[0:nd]
        didx = bufs[nd:2 * nd]
        rows = bufs[2 * nd:3 * nd]
        sg = bufs[3 * nd:4 * nd]
        si = bufs[4 * nd:5 * nd]
        ss = bufs[5 * nd:6 * nd]

        cid = lax.axis_index("c")
        sid = lax.axis_index("s")
        wid = sid * NC + cid

        pltpu.sync_copy(zeros_hbm,
                        acc_sh.at[pl.ds(sid * rows_per_sub, rows_per_sub)])
        plsc.subcore_barrier()

        base_edge = wid * edges_per_tile
        max_off = e - CH

        def idx_start(j, b):
            off = jnp.minimum(base_edge + j * CH, max_off)
            pltpu.async_copy(src_hbm.at[pl.ds(off, CH)], sidx[b], si[b])
            pltpu.async_copy(dst_hbm.at[pl.ds(off, CH)], didx[b], si[b])

        def idx_wait(b):
            pltpu.make_async_copy(src_hbm.at[pl.ds(0, CH)], sidx[b],
                                  si[b]).wait()
            pltpu.make_async_copy(dst_hbm.at[pl.ds(0, CH)], didx[b],
                                  si[b]).wait()

        def gather_start(b):
            pltpu.async_copy(table_hbm.at[sidx[b]], rows[b], sg[b])

        def gather_wait(b):
            pltpu.make_async_copy(table_hbm.at[sidx[b]], rows[b],
                                  sg[b]).wait()

        def scatter_start(b):
            pltpu.async_copy(rows[b], acc_sh.at[didx[b]], ss[b], add=True)

        def scatter_wait(b):
            pltpu.make_async_copy(rows[b], acc_sh.at[didx[b]], ss[b]).wait()

        for b in (2, 3):
            pltpu.sync_copy(zeros_hbm.at[pl.ds(0, CH)], rows[b])
            pltpu.sync_copy(dst_hbm.at[pl.ds(0, CH)], didx[b])
            scatter_start(b)
        idx_start(0, 0)
        idx_start(1, 1)
        idx_wait(0)
        gather_start(0)

        def quad(i, carry):
            for k in range(nd):
                c = nd * i + k
                b, b1, b2 = k, (k + 1) % nd, (k + 2) % nd
                idx_wait(b1)
                gather_start(b1)
                gather_wait(b)
                scatter_start(b)
                scatter_wait(b2)
                idx_start(c + 2, b2)
            return carry

        nloop = (nchunk - 2) // nd
        lax.fori_loop(0, nloop, quad, 0)

        idx_wait(1)
        gather_start(1)
        gather_wait(0)
        scatter_start(0)
        gather_wait(1)
        scatter_start(1)
        for b in range(nd):
            scatter_wait(b)

        plsc.subcore_barrier()
        sl = pl.ds(sid * rows_per_sub, rows_per_sub)

        @pl.when(cid == 0)
        def _():
            pltpu.sync_copy(acc_sh.at[sl], out0_hbm.at[sl])

        @pl.when(cid == 1)
        def _():
            pltpu.sync_copy(acc_sh.at[sl], out1_hbm.at[sl])

    return sc_kernel(table, src, dst, zeros)


def _tc_combine(p0, p1, featc, pmat, n, b, t, of2):
    bt = b * t

    def body(p0_ref, p1_ref, fc_ref, pm_ref, out_ref):
        p = p0_ref[...] + p1_ref[...]
        pt = jnp.transpose(p)
        cnt = jnp.maximum(pt[bt:bt + 1, :], 1.0)
        mean = pt[:bt, :] / cnt
        x = jnp.concatenate([fc_ref[...], mean], axis=0)
        y = jnp.dot(pm_ref[...], x,
                    preferred_element_type=jnp.float32)
        out_ref[...] = y.reshape(t, of2, b, NBLK)

    return pl.pallas_call(
        body,
        grid=(-(-n // NBLK),),
        in_specs=[
            pl.BlockSpec((NBLK, ROW), lambda i: (i, 0)),
            pl.BlockSpec((NBLK, ROW), lambda i: (i, 0)),
            pl.BlockSpec((bt, NBLK), lambda i: (0, i)),
            pl.BlockSpec((t * of2 * b, 2 * bt), lambda i: (0, 0)),
        ],
        out_specs=pl.BlockSpec((t, of2, b, NBLK), lambda i: (0, 0, 0, i)),
        out_shape=jax.ShapeDtypeStruct((t, of2, b, n), jnp.float32),
        compiler_params=pltpu.CompilerParams(
            dimension_semantics=("parallel",)),
    )(p0, p1, featc, pmat)


def kernel(features, edge_index, W):
    n, b, t, in_feat = features.shape
    e = edge_index.shape[1]
    bt = b * t

    feat4 = jnp.transpose(features, (1, 2, 3, 0))
    edges3 = edge_index.reshape(2, e // 128, 128).transpose(1, 0, 2)

    table, featc, dst2, src2 = _tc_prep(feat4, edges3, n, b, t)

    npad = -(-n // (NS * 8)) * (NS * 8)
    zeros = jnp.zeros((npad // NS, ROW), jnp.float32)

    p0, p1 = _sc_segment_sum(table, src2.reshape(e), dst2.reshape(e),
                             zeros, n, e)

    w = W.reshape(-1)
    of = w.shape[0]
    of2 = 2 * of
    kk = jnp.arange(bt)
    tt = kk % t
    bb = kk // t
    r_feat = tt * (of2 * b) + jnp.arange(of)[:, None] * b + bb[None, :]
    r_mean = r_feat + of * b
    pmat = jnp.zeros((t * of2 * b, 2 * bt), jnp.float32)
    pmat = pmat.at[r_feat.reshape(-1),
                   jnp.tile(kk, (of,))].set(jnp.repeat(w, bt))
    pmat = pmat.at[r_mean.reshape(-1),
                   jnp.tile(kk + bt, (of,))].set(jnp.repeat(w, bt))

    out_p = _tc_combine(p0, p1, featc, pmat, n, b, t, of2)
    return jnp.transpose(out_p, (3, 2, 0, 1))
